# baseline (speedup 1.0000x reference)
import jax
import jax.numpy as jnp
import numpy as np
from jax.experimental import pallas as pl
from jax.experimental.pallas import tpu as pltpu

_B, _N, _C = 16, 5000, 200
_BPG = 4
_G = _B // _BPG

def _make_argmax_weights() -> np.ndarray:
    import ml_dtypes
    bf16 = ml_dtypes.bfloat16
    ids = np.arange(_C, dtype=np.float32)
    q = ids * ids
    q_hi = q.astype(bf16)
    q_lo = (q - q_hi.astype(np.float32)).astype(bf16)
    return np.stack([np.ones((_C,), bf16), ids.astype(bf16), q_hi, q_lo])


_W_NP = _make_argmax_weights()


def _body(vd_ref, off_ref, w_ref, logits_ref, seg_ref,
          scores_ref, labels_ref, segs_ref, mask_ref):
  g = pl.program_id(0)
  for i in range(_BPG):
    b = g * _BPG + i
    row = pl.ds(b, 1)

    xt = logits_ref[i]
    m = jnp.max(xt, axis=0, keepdims=True)
    scores_ref[row, :] = jax.nn.sigmoid(m)
    onehot = (xt == m).astype(jnp.bfloat16)
    r = jax.lax.dot_general(w_ref[...], onehot, (((1,), (0,)), ((), ())),
                            preferred_element_type=jnp.float32)
    cnt, s = r[0:1], r[1:2]
    q = r[2:3] + r[3:4]
    tie = (s - jnp.sqrt(jnp.maximum(2.0 * q - s * s, 0.0))) * 0.5
    lbl = jnp.where(cnt > 1.5, tie, s)
    labels_ref[row, :] = lbl.astype(jnp.int32)

    off = off_ref[b]
    vd = vd_ref[b]
    st = seg_ref[i]
    c = st[0:1, :]
    half_w = 0.5 * jnp.exp(st[1:2, :])
    t1 = jnp.clip(c - half_w + off, 0.0, vd)
    t2 = jnp.clip(c + half_w + off, 0.0, vd)
    segs_ref[row, 0:1, :] = t1[None]
    segs_ref[row, 1:2, :] = t2[None]
    mask_ref[row, :] = (t2 - t1) > 0.05


@jax.jit
def kernel(pred_logits, pred_segments, video_durations, feature_durations,
           strides, offsets):
    del feature_durations, strides
    lt = jnp.transpose(pred_logits, (0, 2, 1))
    st = jnp.transpose(pred_segments, (0, 2, 1))

    w = jnp.asarray(_W_NP)

    smem_spec = pl.BlockSpec(memory_space=pltpu.SMEM)

    scores, labels, segs2, valid_mask = pl.pallas_call(
        _body,
        grid=(_G,),
        in_specs=[
            smem_spec,
            smem_spec,
            pl.BlockSpec((4, _C), lambda g: (0, 0)),
            pl.BlockSpec((_BPG, _C, _N), lambda g: (g, 0, 0)),
            pl.BlockSpec((_BPG, 2, _N), lambda g: (g, 0, 0)),
        ],
        out_specs=[
            pl.BlockSpec((_B, _N), lambda g: (0, 0)),
            pl.BlockSpec((_B, _N), lambda g: (0, 0)),
            pl.BlockSpec((_B, 2, _N), lambda g: (0, 0, 0)),
            pl.BlockSpec((_B, _N), lambda g: (0, 0)),
        ],
        out_shape=[
            jax.ShapeDtypeStruct((_B, _N), jnp.float32),
            jax.ShapeDtypeStruct((_B, _N), jnp.int32),
            jax.ShapeDtypeStruct((_B, 2, _N), jnp.float32),
            jax.ShapeDtypeStruct((_B, _N), jnp.bool_),
        ],
    )(video_durations, offsets, w, lt, st)

    segments = jnp.transpose(segs2, (0, 2, 1))
    return scores, labels, segments, valid_mask


# speedup vs baseline: 1.0638x; 1.0638x over previous
import jax
import jax.numpy as jnp
import numpy as np
from jax.experimental import pallas as pl
from jax.experimental.pallas import tpu as pltpu

_B, _N, _C = 16, 5000, 200
_BPG = 2
_G = _B // _BPG

def _make_argmax_weights() -> np.ndarray:
    import ml_dtypes
    bf16 = ml_dtypes.bfloat16
    ids = np.arange(_C, dtype=np.float32)
    q = ids * ids
    q_hi = q.astype(bf16)
    q_lo = (q - q_hi.astype(np.float32)).astype(bf16)
    return np.stack([np.ones((_C,), bf16), ids.astype(bf16), q_hi, q_lo])


_W_NP = _make_argmax_weights()


def _body(vd_ref, off_ref, w_ref, logits_ref, seg_ref,
          scores_ref, labels_ref, segs_ref, mask_ref):
  g = pl.program_id(0)
  for i in range(_BPG):
    b = g * _BPG + i
    row = pl.ds(b, 1)

    xt = logits_ref[i]
    m = jnp.max(xt, axis=0, keepdims=True)
    scores_ref[row, :] = jax.nn.sigmoid(m)
    onehot = (xt == m).astype(jnp.bfloat16)
    r = jax.lax.dot_general(w_ref[...], onehot, (((1,), (0,)), ((), ())),
                            preferred_element_type=jnp.float32)
    cnt, s = r[0:1], r[1:2]
    q = r[2:3] + r[3:4]
    tie = (s - jnp.sqrt(jnp.maximum(2.0 * q - s * s, 0.0))) * 0.5
    lbl = jnp.where(cnt > 1.5, tie, s)
    labels_ref[row, :] = lbl.astype(jnp.int32)

    off = off_ref[b]
    vd = vd_ref[b]
    st = seg_ref[i]
    c = st[0:1, :]
    half_w = 0.5 * jnp.exp(st[1:2, :])
    t1 = jnp.clip(c - half_w + off, 0.0, vd)
    t2 = jnp.clip(c + half_w + off, 0.0, vd)
    segs_ref[row, 0:1, :] = t1[None]
    segs_ref[row, 1:2, :] = t2[None]

  @pl.when(g == _G - 1)
  def _store_mask():
    d = segs_ref[:, 1, :] - segs_ref[:, 0, :]
    mask_ref[...] = (d > 0.05).astype(jnp.int8)


@jax.jit
def kernel(pred_logits, pred_segments, video_durations, feature_durations,
           strides, offsets):
    del feature_durations, strides
    lt = jnp.transpose(pred_logits, (0, 2, 1))
    st = jnp.transpose(pred_segments, (0, 2, 1))

    w = jnp.asarray(_W_NP)

    smem_spec = pl.BlockSpec(memory_space=pltpu.SMEM)

    scores, labels, segs2, valid_mask = pl.pallas_call(
        _body,
        grid=(_G,),
        in_specs=[
            smem_spec,
            smem_spec,
            pl.BlockSpec((4, _C), lambda g: (0, 0)),
            pl.BlockSpec((_BPG, _C, _N), lambda g: (g, 0, 0)),
            pl.BlockSpec((_BPG, 2, _N), lambda g: (g, 0, 0)),
        ],
        out_specs=[
            pl.BlockSpec((_B, _N), lambda g: (0, 0)),
            pl.BlockSpec((_B, _N), lambda g: (0, 0)),
            pl.BlockSpec((_B, 2, _N), lambda g: (0, 0, 0)),
            pl.BlockSpec((_B, _N), lambda g: (0, 0)),
        ],
        out_shape=[
            jax.ShapeDtypeStruct((_B, _N), jnp.float32),
            jax.ShapeDtypeStruct((_B, _N), jnp.int32),
            jax.ShapeDtypeStruct((_B, 2, _N), jnp.float32),
            jax.ShapeDtypeStruct((_B, _N), jnp.int8),
        ],
    )(video_durations, offsets, w, lt, st)

    segments = jnp.transpose(segs2, (0, 2, 1))
    return scores, labels, segments, valid_mask.astype(bool)
